# baseline (speedup 1.0000x reference)
import functools

import jax
import jax.numpy as jnp
from jax import lax
from jax.experimental import pallas as pl
from jax.experimental.pallas import tpu as pltpu
from jax.experimental.pallas import tpu_sc as plsc

N = 10000
E = 320000
D = 128
H = 256
HF = 128

CH = 125
ER = E // CH
NSUB = 16
NP = 10240
NROW = NP // NSUB
ER_S1 = ER // NSUB
ER_S0 = ER // (2 * NSUB)
KB = 16

_mesh = plsc.VectorSubcoreMesh(core_axis_name="c", subcore_axis_name="s")


def _sc_degree_body(dst2d, ones80, zeros16, out0, out1, dst_v, ones_v, deg_s):
    c = lax.axis_index("c")
    s = lax.axis_index("s")
    r0 = s * NROW
    pltpu.sync_copy(zeros16.at[pl.ds(r0, NROW)], deg_s.at[pl.ds(r0, NROW)])
    pltpu.sync_copy(ones80, ones_v)
    e0 = c * (ER // 2) + s * ER_S0
    plsc.subcore_barrier()

    def outer(b, carry):
        pltpu.sync_copy(dst2d.at[pl.ds(e0 + b * KB, KB)], dst_v)

        def body(j, carry2):
            pltpu.sync_copy(ones_v, deg_s.at[dst_v.at[j]], add=True)
            return carry2

        return lax.fori_loop(0, KB, body, carry)

    lax.fori_loop(0, ER_S0 // KB, outer, 0)
    plsc.subcore_barrier()

    @pl.when(c == 0)
    def _():
        pltpu.sync_copy(deg_s.at[pl.ds(r0, NROW)], out0.at[pl.ds(r0, NROW)])

    @pl.when(c == 1)
    def _():
        pltpu.sync_copy(deg_s.at[pl.ds(r0, NROW)], out1.at[pl.ds(r0, NROW)])


_sc_degree = functools.partial(
    pl.kernel,
    out_type=[jax.ShapeDtypeStruct((NP, 16), jnp.float32),
              jax.ShapeDtypeStruct((NP, 16), jnp.float32)],
    mesh=_mesh,
    scratch_types=[
        pltpu.VMEM((KB, CH), jnp.int32),
        pltpu.VMEM((CH, 16), jnp.float32),
        pltpu.VMEM_SHARED((NP, 16), jnp.float32),
    ],
    compiler_params=pltpu.CompilerParams(use_tc_tiling_on_sc=False),
)(_sc_degree_body)


def _sc_scatter_body(hta, htb, src2d, dst2d, zeros, outa, outb,
                src_v, dst_v, rows_v0, rows_v1, sem0, sem1, ssem0, ssem1, agg_s):
    c = lax.axis_index("c")
    s = lax.axis_index("s")
    r0 = s * NROW

    def run(tbl, out):
        pltpu.sync_copy(zeros.at[pl.ds(r0, NROW)], agg_s.at[pl.ds(r0, NROW)])
        plsc.subcore_barrier()

        def outer(b, carry):
            e0 = s * ER_S1 + b * KB
            pltpu.sync_copy(src2d.at[pl.ds(e0, KB)], src_v)
            pltpu.sync_copy(dst2d.at[pl.ds(e0, KB)], dst_v)
            pltpu.async_copy(tbl.at[src_v.at[0]], rows_v0, sem0)
            pltpu.async_copy(tbl.at[src_v.at[1]], rows_v1, sem1)

            def pair(p, carry2):
                j0 = 2 * p
                pltpu.make_async_copy(tbl.at[src_v.at[j0]], rows_v0, sem0).wait()
                sc0 = pltpu.async_copy(rows_v0, agg_s.at[dst_v.at[j0]], ssem0,
                                       add=True)
                pltpu.make_async_copy(tbl.at[src_v.at[j0 + 1]], rows_v1, sem1).wait()
                sc1 = pltpu.async_copy(rows_v1, agg_s.at[dst_v.at[j0 + 1]], ssem1,
                                       add=True)
                sc0.wait()
                pltpu.async_copy(tbl.at[src_v.at[j0 + 2]], rows_v0, sem0)
                sc1.wait()
                pltpu.async_copy(tbl.at[src_v.at[j0 + 3]], rows_v1, sem1)
                return carry2

            lax.fori_loop(0, KB // 2 - 1, pair, 0)
            pltpu.make_async_copy(tbl.at[src_v.at[KB - 2]], rows_v0, sem0).wait()
            sc0 = pltpu.async_copy(rows_v0, agg_s.at[dst_v.at[KB - 2]], ssem0,
                                   add=True)
            pltpu.make_async_copy(tbl.at[src_v.at[KB - 1]], rows_v1, sem1).wait()
            sc1 = pltpu.async_copy(rows_v1, agg_s.at[dst_v.at[KB - 1]], ssem1,
                                   add=True)
            sc0.wait()
            sc1.wait()
            return carry

        lax.fori_loop(0, ER_S1 // KB, outer, 0)
        plsc.subcore_barrier()
        pltpu.sync_copy(agg_s.at[pl.ds(r0, NROW)], out.at[pl.ds(r0, NROW)])

    @pl.when(c == 0)
    def _():
        run(hta, outa)

    @pl.when(c == 1)
    def _():
        run(htb, outb)


_sc_scatter = functools.partial(
    pl.kernel,
    out_type=[jax.ShapeDtypeStruct((NP, HF), jnp.float32),
              jax.ShapeDtypeStruct((NP, HF), jnp.float32)],
    mesh=_mesh,
    scratch_types=[
        pltpu.VMEM((KB, CH), jnp.int32),
        pltpu.VMEM((KB, CH), jnp.int32),
        pltpu.VMEM((CH, HF), jnp.float32),
        pltpu.VMEM((CH, HF), jnp.float32),
        pltpu.SemaphoreType.DMA,
        pltpu.SemaphoreType.DMA,
        pltpu.SemaphoreType.DMA,
        pltpu.SemaphoreType.DMA,
        pltpu.VMEM_SHARED((NP, HF), jnp.float32),
    ],
)(_sc_scatter_body)


RB = 400
GRID = N // RB


def _dis_from(deg0_ref, deg1_ref):
    deg = deg0_ref[:, :1] + deg1_ref[:, :1] + 1.0
    return lax.rsqrt(deg)


def _t1_body(x_ref, w1_ref, deg0_ref, deg1_ref, hta_ref, htb_ref):
    dis = _dis_from(deg0_ref, deg1_ref)
    h = jnp.dot(x_ref[...], w1_ref[...], preferred_element_type=jnp.float32)
    ht = h * dis
    hta_ref[...] = ht[:, :HF]
    htb_ref[...] = ht[:, HF:]


def _t2_body(x_ref, agga_ref, aggb_ref, hta_ref, htb_ref, deg0_ref, deg1_ref,
             w2_ref, l1w_ref, l2w_ref, b1_ref, l1b_ref, g1_ref, be1_ref,
             rm1_ref, rv1_ref,
             h1_ref, ht2a_ref, ht2b_ref, skip2_ref):
    dis = _dis_from(deg0_ref, deg1_ref)
    agg = jnp.concatenate([agga_ref[...], aggb_ref[...]], axis=1)
    ht = jnp.concatenate([hta_ref[...], htb_ref[...]], axis=1)
    gcn1 = dis * (agg + ht) + b1_ref[...]
    pre = gcn1 + jnp.dot(x_ref[...], l1w_ref[...],
                         preferred_element_type=jnp.float32) + l1b_ref[...]
    scale1 = g1_ref[...] * lax.rsqrt(rv1_ref[...] + 1e-5)
    shift1 = be1_ref[...] - rm1_ref[...] * scale1
    h1 = jnp.maximum(pre * scale1 + shift1, 0.0)
    h1_ref[...] = h1
    ht2 = dis * jnp.dot(h1, w2_ref[...], preferred_element_type=jnp.float32)
    ht2a_ref[...] = ht2[:, :HF]
    ht2b_ref[...] = ht2[:, HF:]
    skip2_ref[...] = jnp.dot(h1, l2w_ref[...], preferred_element_type=jnp.float32)


def _t3_body(h1_ref, agga_ref, aggb_ref, ht2a_ref, ht2b_ref, skip2_ref,
             deg0_ref, deg1_ref, b2_ref, l2b_ref, g2_ref, be2_ref,
             rm2_ref, rv2_ref, out_ref):
    dis = _dis_from(deg0_ref, deg1_ref)
    agg = jnp.concatenate([agga_ref[...], aggb_ref[...]], axis=1)
    ht = jnp.concatenate([ht2a_ref[...], ht2b_ref[...]], axis=1)
    gcn2 = dis * (agg + ht) + b2_ref[...]
    pre = gcn2 + skip2_ref[...] + l2b_ref[...]
    scale2 = g2_ref[...] * lax.rsqrt(rv2_ref[...] + 1e-5)
    shift2 = be2_ref[...] - rm2_ref[...] * scale2
    out_ref[...] = h1_ref[...] + jnp.maximum(pre * scale2 + shift2, 0.0)


def _rows(width):
    return pl.BlockSpec((RB, width), lambda i: (i, 0))


def _full(r, cdim):
    return pl.BlockSpec((r, cdim), lambda i: (0, 0))


def kernel(x, edge_index, W1, b1, L1W, L1b, g1, be1, rm1, rv1,
           W2, b2, L2W, L2b, g2, be2, rm2, rv2):
    f32 = jnp.float32
    src2d = edge_index[0].reshape(ER, CH)
    dst2d = edge_index[1].reshape(ER, CH)
    zeros = jnp.zeros((NP, HF), f32)
    zeros16 = jnp.zeros((NP, 16), f32)
    ones80 = jnp.ones((CH, 16), f32)
    row1 = lambda v: v.reshape(1, H)

    deg0, deg1 = _sc_degree(dst2d, ones80, zeros16)

    t1 = pl.pallas_call(
        _t1_body,
        grid=(GRID,),
        in_specs=[_rows(D), _full(D, H), _rows(16), _rows(16)],
        out_specs=[_rows(HF), _rows(HF)],
        out_shape=[jax.ShapeDtypeStruct((N, HF), f32)] * 2,
    )
    hta, htb = t1(x, W1, deg0, deg1)

    agg1a, agg1b = _sc_scatter(hta, htb, src2d, dst2d, zeros)

    t2 = pl.pallas_call(
        _t2_body,
        grid=(GRID,),
        in_specs=[_rows(D), _rows(HF), _rows(HF), _rows(HF), _rows(HF),
                  _rows(16), _rows(16),
                  _full(H, H), _full(D, H), _full(H, H)] + [_full(1, H)] * 6,
        out_specs=[_rows(H), _rows(HF), _rows(HF), _rows(H)],
        out_shape=[jax.ShapeDtypeStruct((N, H), f32),
                   jax.ShapeDtypeStruct((N, HF), f32),
                   jax.ShapeDtypeStruct((N, HF), f32),
                   jax.ShapeDtypeStruct((N, H), f32)],
    )
    h1, ht2a, ht2b, skip2 = t2(x, agg1a, agg1b, hta, htb, deg0, deg1,
                               W2, L1W, L2W, row1(b1), row1(L1b), row1(g1),
                               row1(be1), row1(rm1), row1(rv1))

    agg2a, agg2b = _sc_scatter(ht2a, ht2b, src2d, dst2d, zeros)

    t3 = pl.pallas_call(
        _t3_body,
        grid=(GRID,),
        in_specs=[_rows(H), _rows(HF), _rows(HF), _rows(HF), _rows(HF),
                  _rows(H), _rows(16), _rows(16)] + [_full(1, H)] * 6,
        out_specs=_rows(H),
        out_shape=jax.ShapeDtypeStruct((N, H), f32),
    )
    out = t3(h1, agg2a, agg2b, ht2a, ht2b, skip2, deg0, deg1,
             row1(b2), row1(L2b), row1(g2), row1(be2), row1(rm2), row1(rv2))
    return out


# speedup vs baseline: 1.2206x; 1.2206x over previous
import functools

import jax
import jax.numpy as jnp
from jax import lax
from jax.experimental import pallas as pl
from jax.experimental.pallas import tpu as pltpu
from jax.experimental.pallas import tpu_sc as plsc

N = 10000
E = 320000
D = 128
H = 256
HF = 128

CH = 125
ER = E // CH
NSUB = 16
NP = 10240
NROW = NP // NSUB
ER_S1 = ER // NSUB
ER_S0 = ER // (2 * NSUB)
KB = 16

_mesh = plsc.VectorSubcoreMesh(core_axis_name="c", subcore_axis_name="s")


def _sc_degree_body(dst2d, ones80, zeros16, out0, out1, dst_v, ones_v, deg_s):
    c = lax.axis_index("c")
    s = lax.axis_index("s")
    r0 = s * NROW
    pltpu.sync_copy(zeros16.at[pl.ds(r0, NROW)], deg_s.at[pl.ds(r0, NROW)])
    pltpu.sync_copy(ones80, ones_v)
    e0 = c * (ER // 2) + s * ER_S0
    plsc.subcore_barrier()

    def outer(b, carry):
        pltpu.sync_copy(dst2d.at[pl.ds(e0 + b * KB, KB)], dst_v)

        def body(j, carry2):
            pltpu.sync_copy(ones_v, deg_s.at[dst_v.at[j]], add=True)
            return carry2

        return lax.fori_loop(0, KB, body, carry)

    lax.fori_loop(0, ER_S0 // KB, outer, 0)
    plsc.subcore_barrier()

    @pl.when(c == 0)
    def _():
        pltpu.sync_copy(deg_s.at[pl.ds(r0, NROW)], out0.at[pl.ds(r0, NROW)])

    @pl.when(c == 1)
    def _():
        pltpu.sync_copy(deg_s.at[pl.ds(r0, NROW)], out1.at[pl.ds(r0, NROW)])


_sc_degree = functools.partial(
    pl.kernel,
    out_type=[jax.ShapeDtypeStruct((NP, 16), jnp.float32),
              jax.ShapeDtypeStruct((NP, 16), jnp.float32)],
    mesh=_mesh,
    scratch_types=[
        pltpu.VMEM((KB, CH), jnp.int32),
        pltpu.VMEM((CH, 16), jnp.float32),
        pltpu.VMEM_SHARED((NP, 16), jnp.float32),
    ],
    compiler_params=pltpu.CompilerParams(use_tc_tiling_on_sc=False),
)(_sc_degree_body)


def _sc_scatter_body(hta, htb, src2d, dst2d, zeros, outa, outb,
                src_v, dst_v, rows_v0, rows_v1, sem0, sem1, agg_s):
    c = lax.axis_index("c")
    s = lax.axis_index("s")
    r0 = s * NROW

    def run(tbl, out):
        pltpu.sync_copy(zeros.at[pl.ds(r0, NROW)], agg_s.at[pl.ds(r0, NROW)])
        plsc.subcore_barrier()

        def outer(b, carry):
            e0 = s * ER_S1 + b * KB
            pltpu.sync_copy(src2d.at[pl.ds(e0, KB)], src_v)
            pltpu.sync_copy(dst2d.at[pl.ds(e0, KB)], dst_v)
            pltpu.async_copy(tbl.at[src_v.at[0]], rows_v0, sem0)
            pltpu.async_copy(tbl.at[src_v.at[1]], rows_v1, sem1)

            def pair(p, carry2):
                j0 = 2 * p
                pltpu.make_async_copy(tbl.at[src_v.at[j0]], rows_v0, sem0).wait()
                pltpu.sync_copy(rows_v0, agg_s.at[dst_v.at[j0]], add=True)
                pltpu.async_copy(tbl.at[src_v.at[j0 + 2]], rows_v0, sem0)
                pltpu.make_async_copy(tbl.at[src_v.at[j0 + 1]], rows_v1, sem1).wait()
                pltpu.sync_copy(rows_v1, agg_s.at[dst_v.at[j0 + 1]], add=True)
                pltpu.async_copy(tbl.at[src_v.at[j0 + 3]], rows_v1, sem1)
                return carry2

            lax.fori_loop(0, KB // 2 - 1, pair, 0)
            pltpu.make_async_copy(tbl.at[src_v.at[KB - 2]], rows_v0, sem0).wait()
            pltpu.sync_copy(rows_v0, agg_s.at[dst_v.at[KB - 2]], add=True)
            pltpu.make_async_copy(tbl.at[src_v.at[KB - 1]], rows_v1, sem1).wait()
            pltpu.sync_copy(rows_v1, agg_s.at[dst_v.at[KB - 1]], add=True)
            return carry

        lax.fori_loop(0, ER_S1 // KB, outer, 0)
        plsc.subcore_barrier()
        pltpu.sync_copy(agg_s.at[pl.ds(r0, NROW)], out.at[pl.ds(r0, NROW)])

    @pl.when(c == 0)
    def _():
        run(hta, outa)

    @pl.when(c == 1)
    def _():
        run(htb, outb)


_sc_scatter = functools.partial(
    pl.kernel,
    out_type=[jax.ShapeDtypeStruct((NP, HF), jnp.float32),
              jax.ShapeDtypeStruct((NP, HF), jnp.float32)],
    mesh=_mesh,
    scratch_types=[
        pltpu.VMEM((KB, CH), jnp.int32),
        pltpu.VMEM((KB, CH), jnp.int32),
        pltpu.VMEM((CH, HF), jnp.float32),
        pltpu.VMEM((CH, HF), jnp.float32),
        pltpu.SemaphoreType.DMA,
        pltpu.SemaphoreType.DMA,
        pltpu.VMEM_SHARED((NP, HF), jnp.float32),
    ],
)(_sc_scatter_body)


RB = 400
GRID = N // RB


def _dis_from(deg0_ref, deg1_ref):
    deg = deg0_ref[:, :1] + deg1_ref[:, :1] + 1.0
    return lax.rsqrt(deg)


def _t1_body(x_ref, w1_ref, deg0_ref, deg1_ref, hta_ref, htb_ref):
    dis = _dis_from(deg0_ref, deg1_ref)
    h = jnp.dot(x_ref[...], w1_ref[...], preferred_element_type=jnp.float32)
    ht = h * dis
    hta_ref[...] = ht[:, :HF]
    htb_ref[...] = ht[:, HF:]


def _t2_body(x_ref, agga_ref, aggb_ref, hta_ref, htb_ref, deg0_ref, deg1_ref,
             w2_ref, l1w_ref, l2w_ref, b1_ref, l1b_ref, g1_ref, be1_ref,
             rm1_ref, rv1_ref,
             h1_ref, ht2a_ref, ht2b_ref, skip2_ref):
    dis = _dis_from(deg0_ref, deg1_ref)
    agg = jnp.concatenate([agga_ref[...], aggb_ref[...]], axis=1)
    ht = jnp.concatenate([hta_ref[...], htb_ref[...]], axis=1)
    gcn1 = dis * (agg + ht) + b1_ref[...]
    pre = gcn1 + jnp.dot(x_ref[...], l1w_ref[...],
                         preferred_element_type=jnp.float32) + l1b_ref[...]
    scale1 = g1_ref[...] * lax.rsqrt(rv1_ref[...] + 1e-5)
    shift1 = be1_ref[...] - rm1_ref[...] * scale1
    h1 = jnp.maximum(pre * scale1 + shift1, 0.0)
    h1_ref[...] = h1
    ht2 = dis * jnp.dot(h1, w2_ref[...], preferred_element_type=jnp.float32)
    ht2a_ref[...] = ht2[:, :HF]
    ht2b_ref[...] = ht2[:, HF:]
    skip2_ref[...] = jnp.dot(h1, l2w_ref[...], preferred_element_type=jnp.float32)


def _t3_body(h1_ref, agga_ref, aggb_ref, ht2a_ref, ht2b_ref, skip2_ref,
             deg0_ref, deg1_ref, b2_ref, l2b_ref, g2_ref, be2_ref,
             rm2_ref, rv2_ref, out_ref):
    dis = _dis_from(deg0_ref, deg1_ref)
    agg = jnp.concatenate([agga_ref[...], aggb_ref[...]], axis=1)
    ht = jnp.concatenate([ht2a_ref[...], ht2b_ref[...]], axis=1)
    gcn2 = dis * (agg + ht) + b2_ref[...]
    pre = gcn2 + skip2_ref[...] + l2b_ref[...]
    scale2 = g2_ref[...] * lax.rsqrt(rv2_ref[...] + 1e-5)
    shift2 = be2_ref[...] - rm2_ref[...] * scale2
    out_ref[...] = h1_ref[...] + jnp.maximum(pre * scale2 + shift2, 0.0)


def _rows(width):
    return pl.BlockSpec((RB, width), lambda i: (i, 0))


def _full(r, cdim):
    return pl.BlockSpec((r, cdim), lambda i: (0, 0))


def kernel(x, edge_index, W1, b1, L1W, L1b, g1, be1, rm1, rv1,
           W2, b2, L2W, L2b, g2, be2, rm2, rv2):
    f32 = jnp.float32
    src2d = edge_index[0].reshape(ER, CH)
    dst2d = edge_index[1].reshape(ER, CH)
    zeros = jnp.zeros((NP, HF), f32)
    zeros16 = jnp.zeros((NP, 16), f32)
    ones80 = jnp.ones((CH, 16), f32)
    row1 = lambda v: v.reshape(1, H)

    deg0, deg1 = _sc_degree(dst2d, ones80, zeros16)

    t1 = pl.pallas_call(
        _t1_body,
        grid=(GRID,),
        in_specs=[_rows(D), _full(D, H), _rows(16), _rows(16)],
        out_specs=[_rows(HF), _rows(HF)],
        out_shape=[jax.ShapeDtypeStruct((N, HF), f32)] * 2,
    )
    hta, htb = t1(x, W1, deg0, deg1)

    agg1a, agg1b = _sc_scatter(hta, htb, src2d, dst2d, zeros)

    t2 = pl.pallas_call(
        _t2_body,
        grid=(GRID,),
        in_specs=[_rows(D), _rows(HF), _rows(HF), _rows(HF), _rows(HF),
                  _rows(16), _rows(16),
                  _full(H, H), _full(D, H), _full(H, H)] + [_full(1, H)] * 6,
        out_specs=[_rows(H), _rows(HF), _rows(HF), _rows(H)],
        out_shape=[jax.ShapeDtypeStruct((N, H), f32),
                   jax.ShapeDtypeStruct((N, HF), f32),
                   jax.ShapeDtypeStruct((N, HF), f32),
                   jax.ShapeDtypeStruct((N, H), f32)],
    )
    h1, ht2a, ht2b, skip2 = t2(x, agg1a, agg1b, hta, htb, deg0, deg1,
                               W2, L1W, L2W, row1(b1), row1(L1b), row1(g1),
                               row1(be1), row1(rm1), row1(rv1))

    agg2a, agg2b = _sc_scatter(ht2a, ht2b, src2d, dst2d, zeros)

    t3 = pl.pallas_call(
        _t3_body,
        grid=(GRID,),
        in_specs=[_rows(H), _rows(HF), _rows(HF), _rows(HF), _rows(HF),
                  _rows(H), _rows(16), _rows(16)] + [_full(1, H)] * 6,
        out_specs=_rows(H),
        out_shape=jax.ShapeDtypeStruct((N, H), f32),
    )
    out = t3(h1, agg2a, agg2b, ht2a, ht2b, skip2, deg0, deg1,
             row1(b2), row1(L2b), row1(g2), row1(be2), row1(rm2), row1(rv2))
    return out


# speedup vs baseline: 1.2970x; 1.0626x over previous
import functools

import jax
import jax.numpy as jnp
from jax import lax
from jax.experimental import pallas as pl
from jax.experimental.pallas import tpu as pltpu
from jax.experimental.pallas import tpu_sc as plsc

N = 10000
E = 320000
D = 128
H = 256
HF = 128

CH = 125
ER = E // CH
NSUB = 16
NP = 10240
NROW = NP // NSUB
ER_S1 = ER // NSUB
ER_S0 = ER // (2 * NSUB)
KB = 40

_mesh = plsc.VectorSubcoreMesh(core_axis_name="c", subcore_axis_name="s")


def _sc_degree_body(dst2d, ones80, zeros16, out0, out1, dst_v, ones_v, deg_s):
    c = lax.axis_index("c")
    s = lax.axis_index("s")
    r0 = s * NROW
    pltpu.sync_copy(zeros16.at[pl.ds(r0, NROW)], deg_s.at[pl.ds(r0, NROW)])
    pltpu.sync_copy(ones80, ones_v)
    e0 = c * (ER // 2) + s * ER_S0
    plsc.subcore_barrier()

    def outer(b, carry):
        pltpu.sync_copy(dst2d.at[pl.ds(e0 + b * KB, KB)], dst_v)

        def body(j, carry2):
            pltpu.sync_copy(ones_v, deg_s.at[dst_v.at[j]], add=True)
            return carry2

        return lax.fori_loop(0, KB, body, carry)

    lax.fori_loop(0, ER_S0 // KB, outer, 0)
    plsc.subcore_barrier()

    @pl.when(c == 0)
    def _():
        pltpu.sync_copy(deg_s.at[pl.ds(r0, NROW)], out0.at[pl.ds(r0, NROW)])

    @pl.when(c == 1)
    def _():
        pltpu.sync_copy(deg_s.at[pl.ds(r0, NROW)], out1.at[pl.ds(r0, NROW)])


_sc_degree = functools.partial(
    pl.kernel,
    out_type=[jax.ShapeDtypeStruct((NP, 16), jnp.float32),
              jax.ShapeDtypeStruct((NP, 16), jnp.float32)],
    mesh=_mesh,
    scratch_types=[
        pltpu.VMEM((KB, CH), jnp.int32),
        pltpu.VMEM((CH, 16), jnp.float32),
        pltpu.VMEM_SHARED((NP, 16), jnp.float32),
    ],
    compiler_params=pltpu.CompilerParams(use_tc_tiling_on_sc=False),
)(_sc_degree_body)


def _sc_scatter_body(hta, htb, src2d, dst2d, zeros, outa, outb,
                src_v, dst_v, rows_v0, rows_v1, sem0, sem1, agg_s):
    c = lax.axis_index("c")
    s = lax.axis_index("s")
    r0 = s * NROW

    def run(tbl, out):
        pltpu.sync_copy(zeros.at[pl.ds(r0, NROW)], agg_s.at[pl.ds(r0, NROW)])
        plsc.subcore_barrier()

        def outer(b, carry):
            e0 = s * ER_S1 + b * KB
            pltpu.sync_copy(src2d.at[pl.ds(e0, KB)], src_v)
            pltpu.sync_copy(dst2d.at[pl.ds(e0, KB)], dst_v)
            pltpu.async_copy(tbl.at[src_v.at[0]], rows_v0, sem0)
            pltpu.async_copy(tbl.at[src_v.at[1]], rows_v1, sem1)

            def pair(p, carry2):
                j0 = 2 * p
                pltpu.make_async_copy(tbl.at[src_v.at[j0]], rows_v0, sem0).wait()
                pltpu.sync_copy(rows_v0, agg_s.at[dst_v.at[j0]], add=True)
                pltpu.async_copy(tbl.at[src_v.at[j0 + 2]], rows_v0, sem0)
                pltpu.make_async_copy(tbl.at[src_v.at[j0 + 1]], rows_v1, sem1).wait()
                pltpu.sync_copy(rows_v1, agg_s.at[dst_v.at[j0 + 1]], add=True)
                pltpu.async_copy(tbl.at[src_v.at[j0 + 3]], rows_v1, sem1)
                return carry2

            lax.fori_loop(0, KB // 2 - 1, pair, 0)
            pltpu.make_async_copy(tbl.at[src_v.at[KB - 2]], rows_v0, sem0).wait()
            pltpu.sync_copy(rows_v0, agg_s.at[dst_v.at[KB - 2]], add=True)
            pltpu.make_async_copy(tbl.at[src_v.at[KB - 1]], rows_v1, sem1).wait()
            pltpu.sync_copy(rows_v1, agg_s.at[dst_v.at[KB - 1]], add=True)
            return carry

        lax.fori_loop(0, ER_S1 // KB, outer, 0)
        plsc.subcore_barrier()
        pltpu.sync_copy(agg_s.at[pl.ds(r0, NROW)], out.at[pl.ds(r0, NROW)])

    @pl.when(c == 0)
    def _():
        run(hta, outa)

    @pl.when(c == 1)
    def _():
        run(htb, outb)


_sc_scatter = functools.partial(
    pl.kernel,
    out_type=[jax.ShapeDtypeStruct((NP, HF), jnp.float32),
              jax.ShapeDtypeStruct((NP, HF), jnp.float32)],
    mesh=_mesh,
    scratch_types=[
        pltpu.VMEM((KB, CH), jnp.int32),
        pltpu.VMEM((KB, CH), jnp.int32),
        pltpu.VMEM((CH, HF), jnp.float32),
        pltpu.VMEM((CH, HF), jnp.float32),
        pltpu.SemaphoreType.DMA,
        pltpu.SemaphoreType.DMA,
        pltpu.VMEM_SHARED((NP, HF), jnp.float32),
    ],
)(_sc_scatter_body)


RB = 400
GRID = N // RB


def _dis_from(deg0_ref, deg1_ref):
    deg = deg0_ref[:, :1] + deg1_ref[:, :1] + 1.0
    return lax.rsqrt(deg)


def _t1_body(x_ref, w1_ref, deg0_ref, deg1_ref, hta_ref, htb_ref):
    dis = _dis_from(deg0_ref, deg1_ref)
    h = jnp.dot(x_ref[...], w1_ref[...], preferred_element_type=jnp.float32)
    ht = h * dis
    hta_ref[...] = ht[:, :HF]
    htb_ref[...] = ht[:, HF:]


def _t2_body(x_ref, agga_ref, aggb_ref, hta_ref, htb_ref, deg0_ref, deg1_ref,
             w2_ref, l1w_ref, l2w_ref, b1_ref, l1b_ref, g1_ref, be1_ref,
             rm1_ref, rv1_ref,
             h1_ref, ht2a_ref, ht2b_ref, skip2_ref):
    dis = _dis_from(deg0_ref, deg1_ref)
    agg = jnp.concatenate([agga_ref[...], aggb_ref[...]], axis=1)
    ht = jnp.concatenate([hta_ref[...], htb_ref[...]], axis=1)
    gcn1 = dis * (agg + ht) + b1_ref[...]
    pre = gcn1 + jnp.dot(x_ref[...], l1w_ref[...],
                         preferred_element_type=jnp.float32) + l1b_ref[...]
    scale1 = g1_ref[...] * lax.rsqrt(rv1_ref[...] + 1e-5)
    shift1 = be1_ref[...] - rm1_ref[...] * scale1
    h1 = jnp.maximum(pre * scale1 + shift1, 0.0)
    h1_ref[...] = h1
    ht2 = dis * jnp.dot(h1, w2_ref[...], preferred_element_type=jnp.float32)
    ht2a_ref[...] = ht2[:, :HF]
    ht2b_ref[...] = ht2[:, HF:]
    skip2_ref[...] = jnp.dot(h1, l2w_ref[...], preferred_element_type=jnp.float32)


def _t3_body(h1_ref, agga_ref, aggb_ref, ht2a_ref, ht2b_ref, skip2_ref,
             deg0_ref, deg1_ref, b2_ref, l2b_ref, g2_ref, be2_ref,
             rm2_ref, rv2_ref, out_ref):
    dis = _dis_from(deg0_ref, deg1_ref)
    agg = jnp.concatenate([agga_ref[...], aggb_ref[...]], axis=1)
    ht = jnp.concatenate([ht2a_ref[...], ht2b_ref[...]], axis=1)
    gcn2 = dis * (agg + ht) + b2_ref[...]
    pre = gcn2 + skip2_ref[...] + l2b_ref[...]
    scale2 = g2_ref[...] * lax.rsqrt(rv2_ref[...] + 1e-5)
    shift2 = be2_ref[...] - rm2_ref[...] * scale2
    out_ref[...] = h1_ref[...] + jnp.maximum(pre * scale2 + shift2, 0.0)


def _rows(width):
    return pl.BlockSpec((RB, width), lambda i: (i, 0))


def _full(r, cdim):
    return pl.BlockSpec((r, cdim), lambda i: (0, 0))


def kernel(x, edge_index, W1, b1, L1W, L1b, g1, be1, rm1, rv1,
           W2, b2, L2W, L2b, g2, be2, rm2, rv2):
    f32 = jnp.float32
    src2d = edge_index[0].reshape(ER, CH)
    dst2d = edge_index[1].reshape(ER, CH)
    zeros = jnp.zeros((NP, HF), f32)
    zeros16 = jnp.zeros((NP, 16), f32)
    ones80 = jnp.ones((CH, 16), f32)
    row1 = lambda v: v.reshape(1, H)

    deg0, deg1 = _sc_degree(dst2d, ones80, zeros16)

    t1 = pl.pallas_call(
        _t1_body,
        grid=(GRID,),
        in_specs=[_rows(D), _full(D, H), _rows(16), _rows(16)],
        out_specs=[_rows(HF), _rows(HF)],
        out_shape=[jax.ShapeDtypeStruct((N, HF), f32)] * 2,
    )
    hta, htb = t1(x, W1, deg0, deg1)

    agg1a, agg1b = _sc_scatter(hta, htb, src2d, dst2d, zeros)

    t2 = pl.pallas_call(
        _t2_body,
        grid=(GRID,),
        in_specs=[_rows(D), _rows(HF), _rows(HF), _rows(HF), _rows(HF),
                  _rows(16), _rows(16),
                  _full(H, H), _full(D, H), _full(H, H)] + [_full(1, H)] * 6,
        out_specs=[_rows(H), _rows(HF), _rows(HF), _rows(H)],
        out_shape=[jax.ShapeDtypeStruct((N, H), f32),
                   jax.ShapeDtypeStruct((N, HF), f32),
                   jax.ShapeDtypeStruct((N, HF), f32),
                   jax.ShapeDtypeStruct((N, H), f32)],
    )
    h1, ht2a, ht2b, skip2 = t2(x, agg1a, agg1b, hta, htb, deg0, deg1,
                               W2, L1W, L2W, row1(b1), row1(L1b), row1(g1),
                               row1(be1), row1(rm1), row1(rv1))

    agg2a, agg2b = _sc_scatter(ht2a, ht2b, src2d, dst2d, zeros)

    t3 = pl.pallas_call(
        _t3_body,
        grid=(GRID,),
        in_specs=[_rows(H), _rows(HF), _rows(HF), _rows(HF), _rows(HF),
                  _rows(H), _rows(16), _rows(16)] + [_full(1, H)] * 6,
        out_specs=_rows(H),
        out_shape=jax.ShapeDtypeStruct((N, H), f32),
    )
    out = t3(h1, agg2a, agg2b, ht2a, ht2b, skip2, deg0, deg1,
             row1(b2), row1(L2b), row1(g2), row1(be2), row1(rm2), row1(rv2))
    return out


# speedup vs baseline: 1.3214x; 1.0187x over previous
import functools

import jax
import jax.numpy as jnp
from jax import lax
from jax.experimental import pallas as pl
from jax.experimental.pallas import tpu as pltpu
from jax.experimental.pallas import tpu_sc as plsc

N = 10000
E = 320000
D = 128
H = 256
HF = 128

CH = 125
ER = E // CH
NSUB = 16
NP = 10240
NROW = NP // NSUB
ER_S1 = ER // NSUB
ER_S0 = ER // (2 * NSUB)
KB = 40

_mesh = plsc.VectorSubcoreMesh(core_axis_name="c", subcore_axis_name="s")


def _sc_degree_body(dst2d, ones80, zeros16, out0, out1, dst_v, ones_v, deg_s):
    c = lax.axis_index("c")
    s = lax.axis_index("s")
    r0 = s * NROW
    pltpu.sync_copy(zeros16.at[pl.ds(r0, NROW)], deg_s.at[pl.ds(r0, NROW)])
    pltpu.sync_copy(ones80, ones_v)
    e0 = c * (ER // 2) + s * ER_S0
    plsc.subcore_barrier()

    def outer(b, carry):
        pltpu.sync_copy(dst2d.at[pl.ds(e0 + b * KB, KB)], dst_v)

        def body(j, carry2):
            pltpu.sync_copy(ones_v, deg_s.at[dst_v.at[j]], add=True)
            return carry2

        return lax.fori_loop(0, KB, body, carry)

    lax.fori_loop(0, ER_S0 // KB, outer, 0)
    plsc.subcore_barrier()

    @pl.when(c == 0)
    def _():
        pltpu.sync_copy(deg_s.at[pl.ds(r0, NROW)], out0.at[pl.ds(r0, NROW)])

    @pl.when(c == 1)
    def _():
        pltpu.sync_copy(deg_s.at[pl.ds(r0, NROW)], out1.at[pl.ds(r0, NROW)])


_sc_degree = functools.partial(
    pl.kernel,
    out_type=[jax.ShapeDtypeStruct((NP, 16), jnp.float32),
              jax.ShapeDtypeStruct((NP, 16), jnp.float32)],
    mesh=_mesh,
    scratch_types=[
        pltpu.VMEM((KB, CH), jnp.int32),
        pltpu.VMEM((CH, 16), jnp.float32),
        pltpu.VMEM_SHARED((NP, 16), jnp.float32),
    ],
    compiler_params=pltpu.CompilerParams(use_tc_tiling_on_sc=False),
)(_sc_degree_body)


def _sc_scatter_body(hta, htb, src2d, dst2d, outa, outb,
                src_v, dst_v, rows_v0, rows_v1, sem0, sem1, agg_s):
    c = lax.axis_index("c")
    s = lax.axis_index("s")
    r0 = s * NROW

    def run(tbl, out):
        pltpu.sync_copy(tbl.at[pl.ds(r0, NROW)], agg_s.at[pl.ds(r0, NROW)])
        plsc.subcore_barrier()

        def outer(b, carry):
            e0 = s * ER_S1 + b * KB
            pltpu.sync_copy(src2d.at[pl.ds(e0, KB)], src_v)
            pltpu.sync_copy(dst2d.at[pl.ds(e0, KB)], dst_v)
            pltpu.async_copy(tbl.at[src_v.at[0]], rows_v0, sem0)
            pltpu.async_copy(tbl.at[src_v.at[1]], rows_v1, sem1)

            def pair(p, carry2):
                j0 = 2 * p
                pltpu.make_async_copy(tbl.at[src_v.at[j0]], rows_v0, sem0).wait()
                pltpu.sync_copy(rows_v0, agg_s.at[dst_v.at[j0]], add=True)
                pltpu.async_copy(tbl.at[src_v.at[j0 + 2]], rows_v0, sem0)
                pltpu.make_async_copy(tbl.at[src_v.at[j0 + 1]], rows_v1, sem1).wait()
                pltpu.sync_copy(rows_v1, agg_s.at[dst_v.at[j0 + 1]], add=True)
                pltpu.async_copy(tbl.at[src_v.at[j0 + 3]], rows_v1, sem1)
                return carry2

            lax.fori_loop(0, KB // 2 - 1, pair, 0)
            pltpu.make_async_copy(tbl.at[src_v.at[KB - 2]], rows_v0, sem0).wait()
            pltpu.sync_copy(rows_v0, agg_s.at[dst_v.at[KB - 2]], add=True)
            pltpu.make_async_copy(tbl.at[src_v.at[KB - 1]], rows_v1, sem1).wait()
            pltpu.sync_copy(rows_v1, agg_s.at[dst_v.at[KB - 1]], add=True)
            return carry

        lax.fori_loop(0, ER_S1 // KB, outer, 0)
        plsc.subcore_barrier()
        pltpu.sync_copy(agg_s.at[pl.ds(r0, NROW)], out.at[pl.ds(r0, NROW)])

    @pl.when(c == 0)
    def _():
        run(hta, outa)

    @pl.when(c == 1)
    def _():
        run(htb, outb)


_sc_scatter = functools.partial(
    pl.kernel,
    out_type=[jax.ShapeDtypeStruct((NP, HF), jnp.float32),
              jax.ShapeDtypeStruct((NP, HF), jnp.float32)],
    mesh=_mesh,
    scratch_types=[
        pltpu.VMEM((KB, CH), jnp.int32),
        pltpu.VMEM((KB, CH), jnp.int32),
        pltpu.VMEM((CH, HF), jnp.float32),
        pltpu.VMEM((CH, HF), jnp.float32),
        pltpu.SemaphoreType.DMA,
        pltpu.SemaphoreType.DMA,
        pltpu.VMEM_SHARED((NP, HF), jnp.float32),
    ],
)(_sc_scatter_body)


RB = 512
GRID = NP // RB
RB3 = 400
GRID3 = N // RB3


def _dis_from(deg0_ref, deg1_ref):
    deg = deg0_ref[:, :1] + deg1_ref[:, :1] + 1.0
    return lax.rsqrt(deg)


def _t1_body(x_ref, w1_ref, deg0_ref, deg1_ref, hta_ref, htb_ref):
    dis = _dis_from(deg0_ref, deg1_ref)
    h = jnp.dot(x_ref[...], w1_ref[...], preferred_element_type=jnp.float32)
    ht = h * dis
    hta_ref[...] = ht[:, :HF]
    htb_ref[...] = ht[:, HF:]


def _t2_body(x_ref, agga_ref, aggb_ref, deg0_ref, deg1_ref,
             w2_ref, l1w_ref, l2w_ref, b1_ref, l1b_ref, g1_ref, be1_ref,
             rm1_ref, rv1_ref,
             h1_ref, ht2a_ref, ht2b_ref, skip2_ref):
    dis = _dis_from(deg0_ref, deg1_ref)
    agg = jnp.concatenate([agga_ref[...], aggb_ref[...]], axis=1)
    gcn1 = dis * agg + b1_ref[...]
    pre = gcn1 + jnp.dot(x_ref[...], l1w_ref[...],
                         preferred_element_type=jnp.float32) + l1b_ref[...]
    scale1 = g1_ref[...] * lax.rsqrt(rv1_ref[...] + 1e-5)
    shift1 = be1_ref[...] - rm1_ref[...] * scale1
    h1 = jnp.maximum(pre * scale1 + shift1, 0.0)
    h1_ref[...] = h1
    ht2 = dis * jnp.dot(h1, w2_ref[...], preferred_element_type=jnp.float32)
    ht2a_ref[...] = ht2[:, :HF]
    ht2b_ref[...] = ht2[:, HF:]
    skip2_ref[...] = jnp.dot(h1, l2w_ref[...], preferred_element_type=jnp.float32)


def _t3_body(h1_ref, agga_ref, aggb_ref, skip2_ref,
             deg0_ref, deg1_ref, b2_ref, l2b_ref, g2_ref, be2_ref,
             rm2_ref, rv2_ref, out_ref):
    dis = _dis_from(deg0_ref, deg1_ref)
    agg = jnp.concatenate([agga_ref[...], aggb_ref[...]], axis=1)
    gcn2 = dis * agg + b2_ref[...]
    pre = gcn2 + skip2_ref[...] + l2b_ref[...]
    scale2 = g2_ref[...] * lax.rsqrt(rv2_ref[...] + 1e-5)
    shift2 = be2_ref[...] - rm2_ref[...] * scale2
    out_ref[...] = h1_ref[...] + jnp.maximum(pre * scale2 + shift2, 0.0)


def _rows(width, rb=RB):
    return pl.BlockSpec((rb, width), lambda i: (i, 0))


def _full(r, cdim):
    return pl.BlockSpec((r, cdim), lambda i: (0, 0))


def kernel(x, edge_index, W1, b1, L1W, L1b, g1, be1, rm1, rv1,
           W2, b2, L2W, L2b, g2, be2, rm2, rv2):
    f32 = jnp.float32
    src2d = edge_index[0].reshape(ER, CH)
    dst2d = edge_index[1].reshape(ER, CH)
    zeros16 = jnp.zeros((NP, 16), f32)
    ones80 = jnp.ones((CH, 16), f32)
    xp = jnp.concatenate([x, jnp.zeros((NP - N, D), f32)], axis=0)
    row1 = lambda v: v.reshape(1, H)

    deg0, deg1 = _sc_degree(dst2d, ones80, zeros16)

    t1 = pl.pallas_call(
        _t1_body,
        grid=(GRID,),
        in_specs=[_rows(D), _full(D, H), _rows(16), _rows(16)],
        out_specs=[_rows(HF), _rows(HF)],
        out_shape=[jax.ShapeDtypeStruct((NP, HF), f32)] * 2,
    )
    hta, htb = t1(xp, W1, deg0, deg1)

    agg1a, agg1b = _sc_scatter(hta, htb, src2d, dst2d)

    t2 = pl.pallas_call(
        _t2_body,
        grid=(GRID,),
        in_specs=[_rows(D), _rows(HF), _rows(HF),
                  _rows(16), _rows(16),
                  _full(H, H), _full(D, H), _full(H, H)] + [_full(1, H)] * 6,
        out_specs=[_rows(H), _rows(HF), _rows(HF), _rows(H)],
        out_shape=[jax.ShapeDtypeStruct((NP, H), f32),
                   jax.ShapeDtypeStruct((NP, HF), f32),
                   jax.ShapeDtypeStruct((NP, HF), f32),
                   jax.ShapeDtypeStruct((NP, H), f32)],
    )
    h1, ht2a, ht2b, skip2 = t2(xp, agg1a, agg1b, deg0, deg1,
                               W2, L1W, L2W, row1(b1), row1(L1b), row1(g1),
                               row1(be1), row1(rm1), row1(rv1))

    agg2a, agg2b = _sc_scatter(ht2a, ht2b, src2d, dst2d)

    t3 = pl.pallas_call(
        _t3_body,
        grid=(GRID3,),
        in_specs=[_rows(H, RB3), _rows(HF, RB3), _rows(HF, RB3), _rows(H, RB3),
                  _rows(16, RB3), _rows(16, RB3)] + [_full(1, H)] * 6,
        out_specs=_rows(H, RB3),
        out_shape=jax.ShapeDtypeStruct((N, H), f32),
    )
    out = t3(h1, agg2a, agg2b, skip2, deg0, deg1,
             row1(b2), row1(L2b), row1(g2), row1(be2), row1(rm2), row1(rv2))
    return out


# speedup vs baseline: 1.3298x; 1.0064x over previous
import functools

import jax
import jax.numpy as jnp
from jax import lax
from jax.experimental import pallas as pl
from jax.experimental.pallas import tpu as pltpu
from jax.experimental.pallas import tpu_sc as plsc

N = 10000
E = 320000
D = 128
H = 256
HF = 128

CH = 125
ER = E // CH
NSUB = 16
NP = 10240
NROW = NP // NSUB
ER_S1 = ER // NSUB
ER_S0 = ER // (2 * NSUB)
KB = 40

_mesh = plsc.VectorSubcoreMesh(core_axis_name="c", subcore_axis_name="s")


def _sc_degree_body(dst2d, ones80, zeros16, out0, out1, dst_v, ones_v, deg_s):
    c = lax.axis_index("c")
    s = lax.axis_index("s")
    r0 = s * NROW
    pltpu.sync_copy(zeros16.at[pl.ds(r0, NROW)], deg_s.at[pl.ds(r0, NROW)])
    pltpu.sync_copy(ones80, ones_v)
    e0 = c * (ER // 2) + s * ER_S0
    plsc.subcore_barrier()

    def outer(b, carry):
        pltpu.sync_copy(dst2d.at[pl.ds(e0 + b * KB, KB)], dst_v)

        def body(j, carry2):
            pltpu.sync_copy(ones_v, deg_s.at[dst_v.at[j]], add=True)
            return carry2

        return lax.fori_loop(0, KB, body, carry)

    lax.fori_loop(0, ER_S0 // KB, outer, 0)
    plsc.subcore_barrier()

    @pl.when(c == 0)
    def _():
        pltpu.sync_copy(deg_s.at[pl.ds(r0, NROW)], out0.at[pl.ds(r0, NROW)])

    @pl.when(c == 1)
    def _():
        pltpu.sync_copy(deg_s.at[pl.ds(r0, NROW)], out1.at[pl.ds(r0, NROW)])


_sc_degree = functools.partial(
    pl.kernel,
    out_type=[jax.ShapeDtypeStruct((NP, 16), jnp.float32),
              jax.ShapeDtypeStruct((NP, 16), jnp.float32)],
    mesh=_mesh,
    scratch_types=[
        pltpu.VMEM((KB, CH), jnp.int32),
        pltpu.VMEM((CH, 16), jnp.float32),
        pltpu.VMEM_SHARED((NP, 16), jnp.float32),
    ],
    compiler_params=pltpu.CompilerParams(use_tc_tiling_on_sc=False),
)(_sc_degree_body)


def _sc_scatter_body(hta, htb, src2d, dst2d, outa, outb,
                src_v, dst_v, rows_v0, rows_v1, sem0, sem1, agg_s):
    c = lax.axis_index("c")
    s = lax.axis_index("s")
    r0 = s * NROW

    def run(tbl, out):
        pltpu.sync_copy(tbl.at[pl.ds(r0, NROW)], agg_s.at[pl.ds(r0, NROW)])
        plsc.subcore_barrier()

        def outer(b, carry):
            e0 = s * ER_S1 + b * KB
            pltpu.sync_copy(src2d.at[pl.ds(e0, KB)], src_v)
            pltpu.sync_copy(dst2d.at[pl.ds(e0, KB)], dst_v)
            pltpu.async_copy(tbl.at[src_v.at[0]], rows_v0, sem0)
            pltpu.async_copy(tbl.at[src_v.at[1]], rows_v1, sem1)

            def pair(p, carry2):
                j0 = 2 * p
                pltpu.make_async_copy(tbl.at[src_v.at[j0]], rows_v0, sem0).wait()
                pltpu.sync_copy(rows_v0, agg_s.at[dst_v.at[j0]], add=True)
                pltpu.async_copy(tbl.at[src_v.at[j0 + 2]], rows_v0, sem0)
                pltpu.make_async_copy(tbl.at[src_v.at[j0 + 1]], rows_v1, sem1).wait()
                pltpu.sync_copy(rows_v1, agg_s.at[dst_v.at[j0 + 1]], add=True)
                pltpu.async_copy(tbl.at[src_v.at[j0 + 3]], rows_v1, sem1)
                return carry2

            lax.fori_loop(0, KB // 2 - 1, pair, 0)
            pltpu.make_async_copy(tbl.at[src_v.at[KB - 2]], rows_v0, sem0).wait()
            pltpu.sync_copy(rows_v0, agg_s.at[dst_v.at[KB - 2]], add=True)
            pltpu.make_async_copy(tbl.at[src_v.at[KB - 1]], rows_v1, sem1).wait()
            pltpu.sync_copy(rows_v1, agg_s.at[dst_v.at[KB - 1]], add=True)
            return carry

        lax.fori_loop(0, ER_S1 // KB, outer, 0)
        plsc.subcore_barrier()
        pltpu.sync_copy(agg_s.at[pl.ds(r0, NROW)], out.at[pl.ds(r0, NROW)])

    @pl.when(c == 0)
    def _():
        run(hta, outa)

    @pl.when(c == 1)
    def _():
        run(htb, outb)


_sc_scatter = functools.partial(
    pl.kernel,
    out_type=[jax.ShapeDtypeStruct((NP, HF), jnp.float32),
              jax.ShapeDtypeStruct((NP, HF), jnp.float32)],
    mesh=_mesh,
    scratch_types=[
        pltpu.VMEM((KB, CH), jnp.int32),
        pltpu.VMEM((KB, CH), jnp.int32),
        pltpu.VMEM((CH, HF), jnp.float32),
        pltpu.VMEM((CH, HF), jnp.float32),
        pltpu.SemaphoreType.DMA,
        pltpu.SemaphoreType.DMA,
        pltpu.VMEM_SHARED((NP, HF), jnp.float32),
    ],
)(_sc_scatter_body)


RB = 512
GRID = NP // RB
RB3 = 400
GRID3 = N // RB3


def _dis_from(deg0_ref, deg1_ref):
    deg = deg0_ref[:, :1] + deg1_ref[:, :1] + 1.0
    return lax.rsqrt(deg)


def _t1_body(x_ref, w1_ref, deg0_ref, deg1_ref, hta_ref, htb_ref):
    dis = _dis_from(deg0_ref, deg1_ref)
    h = jnp.dot(x_ref[...], w1_ref[...], preferred_element_type=jnp.float32)
    ht = h * dis
    hta_ref[...] = ht[:, :HF]
    htb_ref[...] = ht[:, HF:]


def _t2_body(x_ref, agga_ref, aggb_ref, deg0_ref, deg1_ref,
             w2_ref, l1w_ref, l2w_ref, b1_ref, l1b_ref, g1_ref, be1_ref,
             rm1_ref, rv1_ref,
             h1_ref, ht2a_ref, ht2b_ref):
    dis = _dis_from(deg0_ref, deg1_ref)
    agg = jnp.concatenate([agga_ref[...], aggb_ref[...]], axis=1)
    gcn1 = dis * agg + b1_ref[...]
    pre = gcn1 + jnp.dot(x_ref[...], l1w_ref[...],
                         preferred_element_type=jnp.float32) + l1b_ref[...]
    scale1 = g1_ref[...] * lax.rsqrt(rv1_ref[...] + 1e-5)
    shift1 = be1_ref[...] - rm1_ref[...] * scale1
    h1 = jnp.maximum(pre * scale1 + shift1, 0.0)
    h1_ref[...] = h1
    ht2 = dis * jnp.dot(h1, w2_ref[...], preferred_element_type=jnp.float32)
    ht2a_ref[...] = ht2[:, :HF]
    ht2b_ref[...] = ht2[:, HF:]


def _t3_body(h1_ref, agga_ref, aggb_ref, l2w_ref,
             deg0_ref, deg1_ref, b2_ref, l2b_ref, g2_ref, be2_ref,
             rm2_ref, rv2_ref, out_ref):
    dis = _dis_from(deg0_ref, deg1_ref)
    agg = jnp.concatenate([agga_ref[...], aggb_ref[...]], axis=1)
    gcn2 = dis * agg + b2_ref[...]
    pre = gcn2 + jnp.dot(h1_ref[...], l2w_ref[...],
                         preferred_element_type=jnp.float32) + l2b_ref[...]
    scale2 = g2_ref[...] * lax.rsqrt(rv2_ref[...] + 1e-5)
    shift2 = be2_ref[...] - rm2_ref[...] * scale2
    out_ref[...] = h1_ref[...] + jnp.maximum(pre * scale2 + shift2, 0.0)


def _rows(width, rb=RB):
    return pl.BlockSpec((rb, width), lambda i: (i, 0))


def _full(r, cdim):
    return pl.BlockSpec((r, cdim), lambda i: (0, 0))


def kernel(x, edge_index, W1, b1, L1W, L1b, g1, be1, rm1, rv1,
           W2, b2, L2W, L2b, g2, be2, rm2, rv2):
    f32 = jnp.float32
    src2d = edge_index[0].reshape(ER, CH)
    dst2d = edge_index[1].reshape(ER, CH)
    zeros16 = jnp.zeros((NP, 16), f32)
    ones80 = jnp.ones((CH, 16), f32)
    xp = jnp.concatenate([x, jnp.zeros((NP - N, D), f32)], axis=0)
    row1 = lambda v: v.reshape(1, H)

    deg0, deg1 = _sc_degree(dst2d, ones80, zeros16)

    t1 = pl.pallas_call(
        _t1_body,
        grid=(GRID,),
        in_specs=[_rows(D), _full(D, H), _rows(16), _rows(16)],
        out_specs=[_rows(HF), _rows(HF)],
        out_shape=[jax.ShapeDtypeStruct((NP, HF), f32)] * 2,
    )
    hta, htb = t1(xp, W1, deg0, deg1)

    agg1a, agg1b = _sc_scatter(hta, htb, src2d, dst2d)

    t2 = pl.pallas_call(
        _t2_body,
        grid=(GRID,),
        in_specs=[_rows(D), _rows(HF), _rows(HF),
                  _rows(16), _rows(16),
                  _full(H, H), _full(D, H), _full(H, H)] + [_full(1, H)] * 6,
        out_specs=[_rows(H), _rows(HF), _rows(HF)],
        out_shape=[jax.ShapeDtypeStruct((NP, H), f32),
                   jax.ShapeDtypeStruct((NP, HF), f32),
                   jax.ShapeDtypeStruct((NP, HF), f32)],
    )
    h1, ht2a, ht2b = t2(xp, agg1a, agg1b, deg0, deg1,
                               W2, L1W, L2W, row1(b1), row1(L1b), row1(g1),
                               row1(be1), row1(rm1), row1(rv1))

    agg2a, agg2b = _sc_scatter(ht2a, ht2b, src2d, dst2d)

    t3 = pl.pallas_call(
        _t3_body,
        grid=(GRID3,),
        in_specs=[_rows(H, RB3), _rows(HF, RB3), _rows(HF, RB3), _full(H, H),
                  _rows(16, RB3), _rows(16, RB3)] + [_full(1, H)] * 6,
        out_specs=_rows(H, RB3),
        out_shape=jax.ShapeDtypeStruct((N, H), f32),
    )
    out = t3(h1, agg2a, agg2b, L2W, deg0, deg1,
             row1(b2), row1(L2b), row1(g2), row1(be2), row1(rm2), row1(rv2))
    return out


# speedup vs baseline: 1.3758x; 1.0346x over previous
import functools

import jax
import jax.numpy as jnp
from jax import lax
from jax.experimental import pallas as pl
from jax.experimental.pallas import tpu as pltpu
from jax.experimental.pallas import tpu_sc as plsc

N = 10000
E = 320000
D = 128
H = 256
HF = 128

CH = 125
ER = E // CH
NSUB = 16
NP = 10240
NROW = NP // NSUB
ER_S1 = ER // NSUB
ER_S0 = ER // (2 * NSUB)
KB = 40
KBS = 16

_mesh = plsc.VectorSubcoreMesh(core_axis_name="c", subcore_axis_name="s")


def _sc_degree_body(dst2d, ones80, zeros16, out0, out1, dst_v, ones_v, deg_s):
    c = lax.axis_index("c")
    s = lax.axis_index("s")
    r0 = s * NROW
    pltpu.sync_copy(zeros16.at[pl.ds(r0, NROW)], deg_s.at[pl.ds(r0, NROW)])
    pltpu.sync_copy(ones80, ones_v)
    e0 = c * (ER // 2) + s * ER_S0
    plsc.subcore_barrier()

    def outer(b, carry):
        pltpu.sync_copy(dst2d.at[pl.ds(e0 + b * KB, KB)], dst_v)

        def body(j, carry2):
            pltpu.sync_copy(ones_v, deg_s.at[dst_v.at[j]], add=True)
            return carry2

        return lax.fori_loop(0, KB, body, carry)

    lax.fori_loop(0, ER_S0 // KB, outer, 0)
    plsc.subcore_barrier()

    @pl.when(c == 0)
    def _():
        pltpu.sync_copy(deg_s.at[pl.ds(r0, NROW)], out0.at[pl.ds(r0, NROW)])

    @pl.when(c == 1)
    def _():
        pltpu.sync_copy(deg_s.at[pl.ds(r0, NROW)], out1.at[pl.ds(r0, NROW)])


_sc_degree = functools.partial(
    pl.kernel,
    out_type=[jax.ShapeDtypeStruct((NP, 16), jnp.float32),
              jax.ShapeDtypeStruct((NP, 16), jnp.float32)],
    mesh=_mesh,
    scratch_types=[
        pltpu.VMEM((KB, CH), jnp.int32),
        pltpu.VMEM((CH, 16), jnp.float32),
        pltpu.VMEM_SHARED((NP, 16), jnp.float32),
    ],
    compiler_params=pltpu.CompilerParams(use_tc_tiling_on_sc=False),
)(_sc_degree_body)


def _sc_scatter_body(hta, htb, src2d, dst2d, outa, outb,
                src_v0, dst_v0, src_v1, dst_v1, rows_v0, rows_v1,
                sem0, sem1, isem, agg_s):
    c = lax.axis_index("c")
    s = lax.axis_index("s")
    r0 = s * NROW
    NBLK = ER_S1 // KBS
    NSB = NBLK // 2

    def run(tbl, out):
        e_base = s * ER_S1
        pltpu.sync_copy(src2d.at[pl.ds(e_base, KBS)], src_v0)
        pltpu.sync_copy(dst2d.at[pl.ds(e_base, KBS)], dst_v0)
        pltpu.async_copy(tbl.at[src_v0.at[0]], rows_v0, sem0)
        pltpu.async_copy(tbl.at[src_v0.at[1]], rows_v1, sem1)
        pltpu.sync_copy(tbl.at[pl.ds(r0, NROW)], agg_s.at[pl.ds(r0, NROW)])
        plsc.subcore_barrier()

        def block(sv, dv, sv_next, dv_next, e_next, more):
            @pl.when(more)
            def _():
                pltpu.async_copy(src2d.at[pl.ds(e_next, KBS)], sv_next, isem)
                pltpu.async_copy(dst2d.at[pl.ds(e_next, KBS)], dv_next, isem)

            def pair(q, carry2):
                j0 = 2 * q
                pltpu.make_async_copy(tbl.at[sv.at[j0]], rows_v0, sem0).wait()
                pltpu.sync_copy(rows_v0, agg_s.at[dv.at[j0]], add=True)
                pltpu.async_copy(tbl.at[sv.at[j0 + 2]], rows_v0, sem0)
                pltpu.make_async_copy(tbl.at[sv.at[j0 + 1]], rows_v1, sem1).wait()
                pltpu.sync_copy(rows_v1, agg_s.at[dv.at[j0 + 1]], add=True)
                pltpu.async_copy(tbl.at[sv.at[j0 + 3]], rows_v1, sem1)
                return carry2

            lax.fori_loop(0, KBS // 2 - 1, pair, 0)

            @pl.when(more)
            def _():
                pltpu.make_async_copy(src2d.at[pl.ds(e_next, KBS)], sv_next,
                                      isem).wait()
                pltpu.make_async_copy(dst2d.at[pl.ds(e_next, KBS)], dv_next,
                                      isem).wait()
            pltpu.make_async_copy(tbl.at[sv.at[KBS - 2]], rows_v0, sem0).wait()
            pltpu.sync_copy(rows_v0, agg_s.at[dv.at[KBS - 2]], add=True)

            @pl.when(more)
            def _():
                pltpu.async_copy(tbl.at[sv_next.at[0]], rows_v0, sem0)
            pltpu.make_async_copy(tbl.at[sv.at[KBS - 1]], rows_v1, sem1).wait()
            pltpu.sync_copy(rows_v1, agg_s.at[dv.at[KBS - 1]], add=True)

            @pl.when(more)
            def _():
                pltpu.async_copy(tbl.at[sv_next.at[1]], rows_v1, sem1)

        def superblock(p, carry):
            eA = e_base + (2 * p) * KBS
            block(src_v0, dst_v0, src_v1, dst_v1, eA + KBS, True)
            block(src_v1, dst_v1, src_v0, dst_v0, eA + 2 * KBS, p < NSB - 1)
            return carry

        lax.fori_loop(0, NSB, superblock, 0)
        plsc.subcore_barrier()
        pltpu.sync_copy(agg_s.at[pl.ds(r0, NROW)], out.at[pl.ds(r0, NROW)])

    @pl.when(c == 0)
    def _():
        run(hta, outa)

    @pl.when(c == 1)
    def _():
        run(htb, outb)


_sc_scatter = functools.partial(
    pl.kernel,
    out_type=[jax.ShapeDtypeStruct((NP, HF), jnp.float32),
              jax.ShapeDtypeStruct((NP, HF), jnp.float32)],
    mesh=_mesh,
    scratch_types=[
        pltpu.VMEM((KBS, CH), jnp.int32),
        pltpu.VMEM((KBS, CH), jnp.int32),
        pltpu.VMEM((KBS, CH), jnp.int32),
        pltpu.VMEM((KBS, CH), jnp.int32),
        pltpu.VMEM((CH, HF), jnp.float32),
        pltpu.VMEM((CH, HF), jnp.float32),
        pltpu.SemaphoreType.DMA,
        pltpu.SemaphoreType.DMA,
        pltpu.SemaphoreType.DMA,
        pltpu.VMEM_SHARED((NP, HF), jnp.float32),
    ],
)(_sc_scatter_body)


RB = 512
GRID = NP // RB
RB3 = 400
GRID3 = N // RB3


def _dis_from(deg0_ref, deg1_ref):
    deg = deg0_ref[:, :1] + deg1_ref[:, :1] + 1.0
    return lax.rsqrt(deg)


def _t1_body(x_ref, w1_ref, deg0_ref, deg1_ref, hta_ref, htb_ref):
    dis = _dis_from(deg0_ref, deg1_ref)
    h = jnp.dot(x_ref[...], w1_ref[...], preferred_element_type=jnp.float32)
    ht = h * dis
    hta_ref[...] = ht[:, :HF]
    htb_ref[...] = ht[:, HF:]


def _t2_body(x_ref, agga_ref, aggb_ref, deg0_ref, deg1_ref,
             w2_ref, l1w_ref, l2w_ref, b1_ref, l1b_ref, g1_ref, be1_ref,
             rm1_ref, rv1_ref,
             h1_ref, ht2a_ref, ht2b_ref):
    dis = _dis_from(deg0_ref, deg1_ref)
    agg = jnp.concatenate([agga_ref[...], aggb_ref[...]], axis=1)
    gcn1 = dis * agg + b1_ref[...]
    pre = gcn1 + jnp.dot(x_ref[...], l1w_ref[...],
                         preferred_element_type=jnp.float32) + l1b_ref[...]
    scale1 = g1_ref[...] * lax.rsqrt(rv1_ref[...] + 1e-5)
    shift1 = be1_ref[...] - rm1_ref[...] * scale1
    h1 = jnp.maximum(pre * scale1 + shift1, 0.0)
    h1_ref[...] = h1
    ht2 = dis * jnp.dot(h1, w2_ref[...], preferred_element_type=jnp.float32)
    ht2a_ref[...] = ht2[:, :HF]
    ht2b_ref[...] = ht2[:, HF:]


def _t3_body(h1_ref, agga_ref, aggb_ref, l2w_ref,
             deg0_ref, deg1_ref, b2_ref, l2b_ref, g2_ref, be2_ref,
             rm2_ref, rv2_ref, out_ref):
    dis = _dis_from(deg0_ref, deg1_ref)
    agg = jnp.concatenate([agga_ref[...], aggb_ref[...]], axis=1)
    gcn2 = dis * agg + b2_ref[...]
    pre = gcn2 + jnp.dot(h1_ref[...], l2w_ref[...],
                         preferred_element_type=jnp.float32) + l2b_ref[...]
    scale2 = g2_ref[...] * lax.rsqrt(rv2_ref[...] + 1e-5)
    shift2 = be2_ref[...] - rm2_ref[...] * scale2
    out_ref[...] = h1_ref[...] + jnp.maximum(pre * scale2 + shift2, 0.0)


def _rows(width, rb=RB):
    return pl.BlockSpec((rb, width), lambda i: (i, 0))


def _full(r, cdim):
    return pl.BlockSpec((r, cdim), lambda i: (0, 0))


def kernel(x, edge_index, W1, b1, L1W, L1b, g1, be1, rm1, rv1,
           W2, b2, L2W, L2b, g2, be2, rm2, rv2):
    f32 = jnp.float32
    src2d = edge_index[0].reshape(ER, CH)
    dst2d = edge_index[1].reshape(ER, CH)
    zeros16 = jnp.zeros((NP, 16), f32)
    ones80 = jnp.ones((CH, 16), f32)
    xp = jnp.concatenate([x, jnp.zeros((NP - N, D), f32)], axis=0)
    row1 = lambda v: v.reshape(1, H)

    deg0, deg1 = _sc_degree(dst2d, ones80, zeros16)

    t1 = pl.pallas_call(
        _t1_body,
        grid=(GRID,),
        in_specs=[_rows(D), _full(D, H), _rows(16), _rows(16)],
        out_specs=[_rows(HF), _rows(HF)],
        out_shape=[jax.ShapeDtypeStruct((NP, HF), f32)] * 2,
    )
    hta, htb = t1(xp, W1, deg0, deg1)

    agg1a, agg1b = _sc_scatter(hta, htb, src2d, dst2d)

    t2 = pl.pallas_call(
        _t2_body,
        grid=(GRID,),
        in_specs=[_rows(D), _rows(HF), _rows(HF),
                  _rows(16), _rows(16),
                  _full(H, H), _full(D, H), _full(H, H)] + [_full(1, H)] * 6,
        out_specs=[_rows(H), _rows(HF), _rows(HF)],
        out_shape=[jax.ShapeDtypeStruct((NP, H), f32),
                   jax.ShapeDtypeStruct((NP, HF), f32),
                   jax.ShapeDtypeStruct((NP, HF), f32)],
    )
    h1, ht2a, ht2b = t2(xp, agg1a, agg1b, deg0, deg1,
                               W2, L1W, L2W, row1(b1), row1(L1b), row1(g1),
                               row1(be1), row1(rm1), row1(rv1))

    agg2a, agg2b = _sc_scatter(ht2a, ht2b, src2d, dst2d)

    t3 = pl.pallas_call(
        _t3_body,
        grid=(GRID3,),
        in_specs=[_rows(H, RB3), _rows(HF, RB3), _rows(HF, RB3), _full(H, H),
                  _rows(16, RB3), _rows(16, RB3)] + [_full(1, H)] * 6,
        out_specs=_rows(H, RB3),
        out_shape=jax.ShapeDtypeStruct((N, H), f32),
    )
    out = t3(h1, agg2a, agg2b, L2W, deg0, deg1,
             row1(b2), row1(L2b), row1(g2), row1(be2), row1(rm2), row1(rv2))
    return out


# speedup vs baseline: 1.3844x; 1.0063x over previous
import functools

import jax
import jax.numpy as jnp
from jax import lax
from jax.experimental import pallas as pl
from jax.experimental.pallas import tpu as pltpu
from jax.experimental.pallas import tpu_sc as plsc

N = 10000
E = 320000
D = 128
H = 256
HF = 128

CH = 125
ER = E // CH
NSUB = 16
NP = 10240
NROW = NP // NSUB
ER_S1 = ER // NSUB
ER_S0 = ER // (2 * NSUB)
KB = 40
KBS = 16

_mesh = plsc.VectorSubcoreMesh(core_axis_name="c", subcore_axis_name="s")


def _sc_degree_body(dst2d, ones80, zeros16, out0, out1, dst_v, ones_v,
                    ssem0, ssem1, deg_s):
    c = lax.axis_index("c")
    s = lax.axis_index("s")
    r0 = s * NROW
    e0 = c * (ER // 2) + s * ER_S0
    pltpu.sync_copy(ones80, ones_v)
    pltpu.sync_copy(dst2d.at[pl.ds(e0, ER_S0)], dst_v)
    pltpu.sync_copy(zeros16.at[pl.ds(r0, NROW)], deg_s.at[pl.ds(r0, NROW)])
    plsc.subcore_barrier()

    pltpu.async_copy(ones_v, deg_s.at[dst_v.at[0]], ssem0, add=True)
    pltpu.async_copy(ones_v, deg_s.at[dst_v.at[1]], ssem1, add=True)

    def pair(q, carry):
        j0 = 2 * q
        pltpu.make_async_copy(ones_v, deg_s.at[dst_v.at[j0]], ssem0).wait()
        pltpu.async_copy(ones_v, deg_s.at[dst_v.at[j0 + 2]], ssem0, add=True)
        pltpu.make_async_copy(ones_v, deg_s.at[dst_v.at[j0 + 1]], ssem1).wait()
        pltpu.async_copy(ones_v, deg_s.at[dst_v.at[j0 + 3]], ssem1, add=True)
        return carry

    lax.fori_loop(0, ER_S0 // 2 - 1, pair, 0)
    pltpu.make_async_copy(ones_v, deg_s.at[dst_v.at[ER_S0 - 2]], ssem0).wait()
    pltpu.make_async_copy(ones_v, deg_s.at[dst_v.at[ER_S0 - 1]], ssem1).wait()
    plsc.subcore_barrier()

    @pl.when(c == 0)
    def _():
        pltpu.sync_copy(deg_s.at[pl.ds(r0, NROW)], out0.at[pl.ds(r0, NROW)])

    @pl.when(c == 1)
    def _():
        pltpu.sync_copy(deg_s.at[pl.ds(r0, NROW)], out1.at[pl.ds(r0, NROW)])


_sc_degree = functools.partial(
    pl.kernel,
    out_type=[jax.ShapeDtypeStruct((NP, 16), jnp.float32),
              jax.ShapeDtypeStruct((NP, 16), jnp.float32)],
    mesh=_mesh,
    scratch_types=[
        pltpu.VMEM((ER_S0, CH), jnp.int32),
        pltpu.VMEM((CH, 16), jnp.float32),
        pltpu.SemaphoreType.DMA,
        pltpu.SemaphoreType.DMA,
        pltpu.VMEM_SHARED((NP, 16), jnp.float32),
    ],
    compiler_params=pltpu.CompilerParams(use_tc_tiling_on_sc=False),
)(_sc_degree_body)


def _sc_scatter_body(hta, htb, src2d, dst2d, outa, outb,
                src_v0, dst_v0, src_v1, dst_v1, rows_v0, rows_v1,
                sem0, sem1, isem, agg_s):
    c = lax.axis_index("c")
    s = lax.axis_index("s")
    r0 = s * NROW
    NBLK = ER_S1 // KBS
    NSB = NBLK // 2

    def run(tbl, out):
        e_base = s * ER_S1
        pltpu.sync_copy(src2d.at[pl.ds(e_base, KBS)], src_v0)
        pltpu.sync_copy(dst2d.at[pl.ds(e_base, KBS)], dst_v0)
        pltpu.async_copy(tbl.at[src_v0.at[0]], rows_v0, sem0)
        pltpu.async_copy(tbl.at[src_v0.at[1]], rows_v1, sem1)
        pltpu.sync_copy(tbl.at[pl.ds(r0, NROW)], agg_s.at[pl.ds(r0, NROW)])
        plsc.subcore_barrier()

        def block(sv, dv, sv_next, dv_next, e_next, more):
            @pl.when(more)
            def _():
                pltpu.async_copy(src2d.at[pl.ds(e_next, KBS)], sv_next, isem)
                pltpu.async_copy(dst2d.at[pl.ds(e_next, KBS)], dv_next, isem)

            def pair(q, carry2):
                j0 = 2 * q
                pltpu.make_async_copy(tbl.at[sv.at[j0]], rows_v0, sem0).wait()
                pltpu.sync_copy(rows_v0, agg_s.at[dv.at[j0]], add=True)
                pltpu.async_copy(tbl.at[sv.at[j0 + 2]], rows_v0, sem0)
                pltpu.make_async_copy(tbl.at[sv.at[j0 + 1]], rows_v1, sem1).wait()
                pltpu.sync_copy(rows_v1, agg_s.at[dv.at[j0 + 1]], add=True)
                pltpu.async_copy(tbl.at[sv.at[j0 + 3]], rows_v1, sem1)
                return carry2

            lax.fori_loop(0, KBS // 2 - 1, pair, 0)

            @pl.when(more)
            def _():
                pltpu.make_async_copy(src2d.at[pl.ds(e_next, KBS)], sv_next,
                                      isem).wait()
                pltpu.make_async_copy(dst2d.at[pl.ds(e_next, KBS)], dv_next,
                                      isem).wait()
            pltpu.make_async_copy(tbl.at[sv.at[KBS - 2]], rows_v0, sem0).wait()
            pltpu.sync_copy(rows_v0, agg_s.at[dv.at[KBS - 2]], add=True)

            @pl.when(more)
            def _():
                pltpu.async_copy(tbl.at[sv_next.at[0]], rows_v0, sem0)
            pltpu.make_async_copy(tbl.at[sv.at[KBS - 1]], rows_v1, sem1).wait()
            pltpu.sync_copy(rows_v1, agg_s.at[dv.at[KBS - 1]], add=True)

            @pl.when(more)
            def _():
                pltpu.async_copy(tbl.at[sv_next.at[1]], rows_v1, sem1)

        def superblock(p, carry):
            eA = e_base + (2 * p) * KBS
            block(src_v0, dst_v0, src_v1, dst_v1, eA + KBS, True)
            block(src_v1, dst_v1, src_v0, dst_v0, eA + 2 * KBS, p < NSB - 1)
            return carry

        lax.fori_loop(0, NSB, superblock, 0)
        plsc.subcore_barrier()
        pltpu.sync_copy(agg_s.at[pl.ds(r0, NROW)], out.at[pl.ds(r0, NROW)])

    @pl.when(c == 0)
    def _():
        run(hta, outa)

    @pl.when(c == 1)
    def _():
        run(htb, outb)


_sc_scatter = functools.partial(
    pl.kernel,
    out_type=[jax.ShapeDtypeStruct((NP, HF), jnp.float32),
              jax.ShapeDtypeStruct((NP, HF), jnp.float32)],
    mesh=_mesh,
    scratch_types=[
        pltpu.VMEM((KBS, CH), jnp.int32),
        pltpu.VMEM((KBS, CH), jnp.int32),
        pltpu.VMEM((KBS, CH), jnp.int32),
        pltpu.VMEM((KBS, CH), jnp.int32),
        pltpu.VMEM((CH, HF), jnp.float32),
        pltpu.VMEM((CH, HF), jnp.float32),
        pltpu.SemaphoreType.DMA,
        pltpu.SemaphoreType.DMA,
        pltpu.SemaphoreType.DMA,
        pltpu.VMEM_SHARED((NP, HF), jnp.float32),
    ],
)(_sc_scatter_body)


RB = 512
GRID = NP // RB
RB3 = 400
GRID3 = N // RB3


def _dis_from(deg0_ref, deg1_ref):
    deg = deg0_ref[:, :1] + deg1_ref[:, :1] + 1.0
    return lax.rsqrt(deg)


def _t1_body(x_ref, w1_ref, deg0_ref, deg1_ref, hta_ref, htb_ref):
    dis = _dis_from(deg0_ref, deg1_ref)
    h = jnp.dot(x_ref[...], w1_ref[...], preferred_element_type=jnp.float32)
    ht = h * dis
    hta_ref[...] = ht[:, :HF]
    htb_ref[...] = ht[:, HF:]


def _t2_body(x_ref, agga_ref, aggb_ref, deg0_ref, deg1_ref,
             w2_ref, l1w_ref, l2w_ref, b1_ref, l1b_ref, g1_ref, be1_ref,
             rm1_ref, rv1_ref,
             h1_ref, ht2a_ref, ht2b_ref):
    dis = _dis_from(deg0_ref, deg1_ref)
    agg = jnp.concatenate([agga_ref[...], aggb_ref[...]], axis=1)
    gcn1 = dis * agg + b1_ref[...]
    pre = gcn1 + jnp.dot(x_ref[...], l1w_ref[...],
                         preferred_element_type=jnp.float32) + l1b_ref[...]
    scale1 = g1_ref[...] * lax.rsqrt(rv1_ref[...] + 1e-5)
    shift1 = be1_ref[...] - rm1_ref[...] * scale1
    h1 = jnp.maximum(pre * scale1 + shift1, 0.0)
    h1_ref[...] = h1
    ht2 = dis * jnp.dot(h1, w2_ref[...], preferred_element_type=jnp.float32)
    ht2a_ref[...] = ht2[:, :HF]
    ht2b_ref[...] = ht2[:, HF:]


def _t3_body(h1_ref, agga_ref, aggb_ref, l2w_ref,
             deg0_ref, deg1_ref, b2_ref, l2b_ref, g2_ref, be2_ref,
             rm2_ref, rv2_ref, out_ref):
    dis = _dis_from(deg0_ref, deg1_ref)
    agg = jnp.concatenate([agga_ref[...], aggb_ref[...]], axis=1)
    gcn2 = dis * agg + b2_ref[...]
    pre = gcn2 + jnp.dot(h1_ref[...], l2w_ref[...],
                         preferred_element_type=jnp.float32) + l2b_ref[...]
    scale2 = g2_ref[...] * lax.rsqrt(rv2_ref[...] + 1e-5)
    shift2 = be2_ref[...] - rm2_ref[...] * scale2
    out_ref[...] = h1_ref[...] + jnp.maximum(pre * scale2 + shift2, 0.0)


def _rows(width, rb=RB):
    return pl.BlockSpec((rb, width), lambda i: (i, 0))


def _full(r, cdim):
    return pl.BlockSpec((r, cdim), lambda i: (0, 0))


def kernel(x, edge_index, W1, b1, L1W, L1b, g1, be1, rm1, rv1,
           W2, b2, L2W, L2b, g2, be2, rm2, rv2):
    f32 = jnp.float32
    src2d = edge_index[0].reshape(ER, CH)
    dst2d = edge_index[1].reshape(ER, CH)
    zeros16 = jnp.zeros((NP, 16), f32)
    ones80 = jnp.ones((CH, 16), f32)
    xp = jnp.concatenate([x, jnp.zeros((NP - N, D), f32)], axis=0)
    row1 = lambda v: v.reshape(1, H)

    deg0, deg1 = _sc_degree(dst2d, ones80, zeros16)

    t1 = pl.pallas_call(
        _t1_body,
        grid=(GRID,),
        in_specs=[_rows(D), _full(D, H), _rows(16), _rows(16)],
        out_specs=[_rows(HF), _rows(HF)],
        out_shape=[jax.ShapeDtypeStruct((NP, HF), f32)] * 2,
    )
    hta, htb = t1(xp, W1, deg0, deg1)

    agg1a, agg1b = _sc_scatter(hta, htb, src2d, dst2d)

    t2 = pl.pallas_call(
        _t2_body,
        grid=(GRID,),
        in_specs=[_rows(D), _rows(HF), _rows(HF),
                  _rows(16), _rows(16),
                  _full(H, H), _full(D, H), _full(H, H)] + [_full(1, H)] * 6,
        out_specs=[_rows(H), _rows(HF), _rows(HF)],
        out_shape=[jax.ShapeDtypeStruct((NP, H), f32),
                   jax.ShapeDtypeStruct((NP, HF), f32),
                   jax.ShapeDtypeStruct((NP, HF), f32)],
    )
    h1, ht2a, ht2b = t2(xp, agg1a, agg1b, deg0, deg1,
                               W2, L1W, L2W, row1(b1), row1(L1b), row1(g1),
                               row1(be1), row1(rm1), row1(rv1))

    agg2a, agg2b = _sc_scatter(ht2a, ht2b, src2d, dst2d)

    t3 = pl.pallas_call(
        _t3_body,
        grid=(GRID3,),
        in_specs=[_rows(H, RB3), _rows(HF, RB3), _rows(HF, RB3), _full(H, H),
                  _rows(16, RB3), _rows(16, RB3)] + [_full(1, H)] * 6,
        out_specs=_rows(H, RB3),
        out_shape=jax.ShapeDtypeStruct((N, H), f32),
    )
    out = t3(h1, agg2a, agg2b, L2W, deg0, deg1,
             row1(b2), row1(L2b), row1(g2), row1(be2), row1(rm2), row1(rv2))
    return out
